# baseline (speedup 1.0000x reference)
import jax
import jax.numpy as jnp
from jax import lax
from jax.experimental import pallas as pl
from jax.experimental.pallas import tpu as pltpu
from jax.experimental.pallas import tpu_sc as plsc

NUM_HEADS = 32
EDGE_HIDDEN = 32
MULTI_HOP_MAX_DIST = 5
NUM_EDGES = 1536
N_GRAPH = 16
N_NODE = 64

_NPAIR = N_GRAPH * N_NODE * N_NODE
_TBL_ROWS = NUM_EDGES + 1
_TROWS_P = 1544
_TSTRIDE = 17
_TSIZE = _TROWS_P * _TSTRIDE

_info = plsc.get_sparse_core_info()
_NC, _NS = _info.num_cores, _info.num_subcores
_NW = _NC * _NS
_CHUNK = _NPAIR // _NW
_GROUPS = _CHUNK // 16


def _tpre_body(e_ref, w_ref, o_ref):
    o_ref[0] = jnp.dot(e_ref[...], w_ref[0], preferred_element_type=jnp.float32)


def _tc_pre(edge_enc_w_pad, w):
    return pl.pallas_call(
        _tpre_body,
        grid=(MULTI_HOP_MAX_DIST,),
        in_specs=[
            pl.BlockSpec((_TROWS_P, EDGE_HIDDEN), lambda d: (0, 0)),
            pl.BlockSpec((1, EDGE_HIDDEN, NUM_HEADS), lambda d: (d, 0, 0)),
        ],
        out_specs=pl.BlockSpec((1, _TROWS_P, NUM_HEADS), lambda d: (d, 0, 0)),
        out_shape=jax.ShapeDtypeStruct(
            (MULTI_HOP_MAX_DIST, _TROWS_P, NUM_HEADS), jnp.float32),
    )(edge_enc_w_pad, w)


_NSLOT = MULTI_HOP_MAX_DIST * 3


def _tr_body(e_ref, o_ref):
    o_ref[...] = e_ref[...].T


def _tc_tr(edge_idx_2d):
    return pl.pallas_call(
        _tr_body,
        grid=(_NW,),
        in_specs=[pl.BlockSpec((_CHUNK, _NSLOT), lambda g: (g, 0))],
        out_specs=pl.BlockSpec((_NSLOT, _CHUNK), lambda g: (0, g)),
        out_shape=jax.ShapeDtypeStruct((_NSLOT, _NPAIR), jnp.int32),
    )(edge_idx_2d)


def _gather_bf(tbl, addr):
    return plsc.bitcast(plsc.load_gather(tbl, [addr]), jnp.bfloat16)


def _sc_body(tp_hbm, idx_hbm, spos_hbm, out_hbm,
             tbl_a, tbl_b, acc, idx6, spos, sem):
    wid = lax.axis_index("s") * _NC + lax.axis_index("c")
    base = wid * _CHUNK

    def round_dmas(da, db):
        cps = [pltpu.async_copy(
            tp_hbm.at[pl.ds(da * _TSIZE, _TSIZE)], tbl_a, sem)]
        if db is not None:
            cps.append(pltpu.async_copy(
                tp_hbm.at[pl.ds(db * _TSIZE, _TSIZE)], tbl_b, sem))
        for k in range(3):
            cps.append(pltpu.async_copy(
                idx_hbm.at[pl.ds(da * 3 + k, 1), pl.ds(base, _CHUNK)],
                idx6.at[pl.ds(k, 1), :], sem))
            if db is not None:
                cps.append(pltpu.async_copy(
                    idx_hbm.at[pl.ds(db * 3 + k, 1), pl.ds(base, _CHUNK)],
                    idx6.at[pl.ds(3 + k, 1), :], sem))
        for c in cps:
            c.wait()

    pltpu.sync_copy(spos_hbm.at[pl.ds(base, _CHUNK)], spos)

    for rnd, (da, db) in enumerate(((0, 1), (2, 3))):
        round_dmas(da, db)

        def eround(g, carry, first=(rnd == 0)):
            p0 = pl.multiple_of(g * 16, 16)
            ia = [idx6[k, pl.ds(p0, 16)] * _TSTRIDE for k in range(3)]
            ib = [idx6[3 + k, pl.ds(p0, 16)] * _TSTRIDE for k in range(3)]
            for cp in range(NUM_HEADS // 2):
                cc = jnp.full((16,), cp, jnp.int32)
                sa = (_gather_bf(tbl_a, ia[0] + cc)
                      + _gather_bf(tbl_a, ia[1] + cc))
                sb = (_gather_bf(tbl_a, ia[2] + cc)
                      + _gather_bf(tbl_b, ib[0] + cc))
                sc = (_gather_bf(tbl_b, ib[1] + cc)
                      + _gather_bf(tbl_b, ib[2] + cc))
                s = sa + sb + sc
                slot = acc.at[cp, pl.ds(p0, 16)]
                if first:
                    slot[...] = plsc.bitcast(s, jnp.int32)
                else:
                    prev = plsc.bitcast(slot[...], jnp.bfloat16)
                    slot[...] = plsc.bitcast(prev + s, jnp.int32)
            return carry

        lax.fori_loop(0, _GROUPS, eround, 0)

    round_dmas(4, None)
    pltpu.sync_copy(tp_hbm.at[pl.ds(5 * _TSIZE, _TSIZE)], tbl_b)

    def sround(g, carry):
        p0 = pl.multiple_of(g * 16, 16)
        ia = [idx6[k, pl.ds(p0, 16)] * _TSTRIDE for k in range(3)]
        sv = spos[pl.ds(p0, 16)]
        sp = jnp.where(sv == 0, 1, sv)
        sp = jnp.where(sp > 1, sp - 1, sp)
        sp = jnp.minimum(sp, MULTI_HOP_MAX_DIST)
        recip = (1.0 / 3.0) / sp.astype(jnp.float32)
        recip2 = plsc.pack(recip, recip, format=plsc.PackFormat.INTERLEAVED)
        svs = sv * _TSTRIDE
        for cp in range(NUM_HEADS // 2):
            cc = jnp.full((16,), cp, jnp.int32)
            s = ((_gather_bf(tbl_a, ia[0] + cc)
                  + _gather_bf(tbl_a, ia[1] + cc))
                 + _gather_bf(tbl_a, ia[2] + cc))
            spb = _gather_bf(tbl_b, svs + cc)
            slot = acc.at[cp, pl.ds(p0, 16)]
            prev = plsc.bitcast(slot[...], jnp.bfloat16)
            slot[...] = plsc.bitcast((prev + s) * recip2 + spb, jnp.int32)
        return carry

    lax.fori_loop(0, _GROUPS, sround, 0)

    pltpu.sync_copy(acc, out_hbm.at[:, pl.ds(base, _CHUNK)])


_sc_kernel = pl.kernel(
    _sc_body,
    out_type=jax.ShapeDtypeStruct((NUM_HEADS // 2, _NPAIR), jnp.int32),
    mesh=plsc.VectorSubcoreMesh(core_axis_name="c", subcore_axis_name="s"),
    compiler_params=pltpu.CompilerParams(needs_layout_passes=False),
    scratch_types=[
        pltpu.VMEM((_TSIZE,), jnp.int32),
        pltpu.VMEM((_TSIZE,), jnp.int32),
        pltpu.VMEM((NUM_HEADS // 2, _CHUNK), jnp.int32),
        pltpu.VMEM((6, _CHUNK), jnp.int32),
        pltpu.VMEM((_CHUNK,), jnp.int32),
        pltpu.SemaphoreType.DMA,
    ],
)


def _asm_body(ab_ref, rt_ref, t_ref, o_ref):
    ab2 = ab_ref[0] * 2.0
    w = rt_ref[...]
    lo = lax.bitcast_convert_type(w << 16, jnp.float32)
    hi = lax.bitcast_convert_type(w & jnp.int32(-65536), jnp.float32)
    inner = jnp.concatenate([lo, hi], axis=0).reshape(
        NUM_HEADS, N_NODE, N_NODE)
    t = t_ref[0]
    n1 = N_NODE + 1
    ii = lax.broadcasted_iota(jnp.int32, (NUM_HEADS, n1, n1), 1)
    jj = lax.broadcasted_iota(jnp.int32, (NUM_HEADS, n1, n1), 2)
    border = (ii == 0) | (jj == 0)
    tb = jnp.where(border,
                   jnp.broadcast_to(t[:, None, None], (NUM_HEADS, n1, n1)),
                   jnp.zeros((NUM_HEADS, n1, n1), jnp.float32))
    z_col = jnp.zeros((NUM_HEADS, N_NODE, 1), jnp.float32)
    z_row = jnp.zeros((NUM_HEADS, 1, n1), jnp.float32)
    padded = jnp.concatenate(
        [z_row, jnp.concatenate([z_col, inner], axis=2)], axis=1)
    o_ref[0] = ab2[None] + tb + padded


def _tc_asm(attn_bias, rt, gtvd_w):
    n1 = N_NODE + 1
    return pl.pallas_call(
        _asm_body,
        grid=(N_GRAPH,),
        in_specs=[
            pl.BlockSpec((1, n1, n1), lambda g: (g, 0, 0)),
            pl.BlockSpec((NUM_HEADS // 2, N_NODE * N_NODE), lambda g: (0, g)),
            pl.BlockSpec((1, NUM_HEADS), lambda g: (0, 0)),
        ],
        out_specs=pl.BlockSpec((1, NUM_HEADS, n1, n1), lambda g: (g, 0, 0, 0)),
        out_shape=jax.ShapeDtypeStruct(
            (N_GRAPH, NUM_HEADS, n1, n1), jnp.float32),
    )(attn_bias, rt, gtvd_w)


def kernel(attn_bias, node_attr, is_molecule, spatial_pos, edge_input,
           spatial_pos_w, gtvd_w, edge_enc_w, edge_dis_w):
    w = edge_dis_w.reshape(-1, EDGE_HIDDEN, NUM_HEADS)[:MULTI_HOP_MAX_DIST]
    eew_pad = jnp.zeros((_TROWS_P, EDGE_HIDDEN), jnp.float32)
    eew_pad = eew_pad.at[:_TBL_ROWS].set(edge_enc_w)
    t5 = _tc_pre(eew_pad, w)

    spw_pad = jnp.zeros((_TROWS_P, NUM_HEADS), jnp.float32)
    spw_pad = spw_pad.at[:spatial_pos_w.shape[0]].set(spatial_pos_w)
    all6 = jnp.concatenate([t5, spw_pad[None]], axis=0)
    lo = lax.bitcast_convert_type(
        all6[..., :16].astype(jnp.bfloat16), jnp.uint16).astype(jnp.uint32)
    hi = lax.bitcast_convert_type(
        all6[..., 16:].astype(jnp.bfloat16), jnp.uint16).astype(jnp.uint32)
    packed = lax.bitcast_convert_type(lo | (hi << 16), jnp.int32)
    packed = jnp.pad(packed, ((0, 0), (0, 0), (0, 1)))
    packed = packed.reshape(6 * _TSIZE)

    idx_t = _tc_tr(edge_input.reshape(_NPAIR, _NSLOT))
    spos_flat = spatial_pos.reshape(_NPAIR)

    rt = _sc_kernel(packed, idx_t, spos_flat)
    return _tc_asm(attn_bias, rt, gtvd_w)


# speedup vs baseline: 1.2051x; 1.2051x over previous
import jax
import jax.numpy as jnp
from jax import lax
from jax.experimental import pallas as pl
from jax.experimental.pallas import tpu as pltpu
from jax.experimental.pallas import tpu_sc as plsc

NUM_HEADS = 32
EDGE_HIDDEN = 32
MULTI_HOP_MAX_DIST = 5
NUM_EDGES = 1536
N_GRAPH = 16
N_NODE = 64

_NPAIR = N_GRAPH * N_NODE * N_NODE
_TBL_ROWS = NUM_EDGES + 1
_TROWS_P = 1544
_TSTRIDE = 17
_TSIZE = _TROWS_P * _TSTRIDE

_info = plsc.get_sparse_core_info()
_NC, _NS = _info.num_cores, _info.num_subcores
_NW = _NC * _NS
_CHUNK = _NPAIR // _NW
_GROUPS = _CHUNK // 16


def _tpre_body(e_ref, w_ref, o_ref):
    o_ref[0] = jnp.dot(e_ref[...], w_ref[0], preferred_element_type=jnp.float32)


def _tc_pre(edge_enc_w_pad, w):
    return pl.pallas_call(
        _tpre_body,
        grid=(MULTI_HOP_MAX_DIST,),
        in_specs=[
            pl.BlockSpec((_TROWS_P, EDGE_HIDDEN), lambda d: (0, 0)),
            pl.BlockSpec((1, EDGE_HIDDEN, NUM_HEADS), lambda d: (d, 0, 0)),
        ],
        out_specs=pl.BlockSpec((1, _TROWS_P, NUM_HEADS), lambda d: (d, 0, 0)),
        out_shape=jax.ShapeDtypeStruct(
            (MULTI_HOP_MAX_DIST, _TROWS_P, NUM_HEADS), jnp.float32),
    )(edge_enc_w_pad, w)


_NSLOT = MULTI_HOP_MAX_DIST * 3


def _tr_body(e_ref, o_ref):
    o_ref[...] = e_ref[...].T


def _tc_tr(edge_idx_2d):
    return pl.pallas_call(
        _tr_body,
        grid=(_NW,),
        in_specs=[pl.BlockSpec((_CHUNK, _NSLOT), lambda g: (g, 0))],
        out_specs=pl.BlockSpec((_NSLOT, _CHUNK), lambda g: (0, g)),
        out_shape=jax.ShapeDtypeStruct((_NSLOT, _NPAIR), jnp.int32),
    )(edge_idx_2d)


def _gather_bf(tbl, addr):
    return plsc.bitcast(plsc.load_gather(tbl, [addr]), jnp.bfloat16)


def _sc_body(tp_hbm, idx_hbm, spos_hbm, out_hbm,
             tbl_a, tbl_b, acc, idx6, spos, sem):
    wid = lax.axis_index("s") * _NC + lax.axis_index("c")
    base = wid * _CHUNK

    def round_dmas(da, db):
        cps = [pltpu.async_copy(
            tp_hbm.at[pl.ds(da * _TSIZE, _TSIZE)], tbl_a, sem)]
        if db is not None:
            cps.append(pltpu.async_copy(
                tp_hbm.at[pl.ds(db * _TSIZE, _TSIZE)], tbl_b, sem))
        for k in range(3):
            cps.append(pltpu.async_copy(
                idx_hbm.at[pl.ds((da * 3 + k) * _NPAIR + base, _CHUNK)],
                idx6.at[pl.ds(k * _CHUNK, _CHUNK)], sem))
            if db is not None:
                cps.append(pltpu.async_copy(
                    idx_hbm.at[pl.ds((db * 3 + k) * _NPAIR + base, _CHUNK)],
                    idx6.at[pl.ds((3 + k) * _CHUNK, _CHUNK)], sem))
        for c in cps:
            c.wait()

    pltpu.sync_copy(spos_hbm.at[pl.ds(base, _CHUNK)], spos)

    for rnd, (da, db) in enumerate(((0, 1), (2, 3))):
        round_dmas(da, db)

        def eround(g, carry, first=(rnd == 0)):
            p0 = pl.multiple_of(g * 16, 16)
            ia = [idx6[pl.ds(k * _CHUNK + p0, 16)] * _TSTRIDE
                  for k in range(3)]
            ib = [idx6[pl.ds((3 + k) * _CHUNK + p0, 16)] * _TSTRIDE
                  for k in range(3)]
            for cp in range(NUM_HEADS // 2):
                cc = jnp.full((16,), cp, jnp.int32)
                sa = (_gather_bf(tbl_a, ia[0] + cc)
                      + _gather_bf(tbl_a, ia[1] + cc))
                sb = (_gather_bf(tbl_a, ia[2] + cc)
                      + _gather_bf(tbl_b, ib[0] + cc))
                sc = (_gather_bf(tbl_b, ib[1] + cc)
                      + _gather_bf(tbl_b, ib[2] + cc))
                s = sa + sb + sc
                slot = acc.at[cp, pl.ds(p0, 16)]
                if first:
                    slot[...] = plsc.bitcast(s, jnp.int32)
                else:
                    prev = plsc.bitcast(slot[...], jnp.bfloat16)
                    slot[...] = plsc.bitcast(prev + s, jnp.int32)
            return carry

        lax.fori_loop(0, _GROUPS, eround, 0)

    round_dmas(4, None)
    pltpu.sync_copy(tp_hbm.at[pl.ds(5 * _TSIZE, _TSIZE)], tbl_b)

    def sround(g, carry):
        p0 = pl.multiple_of(g * 16, 16)
        ia = [idx6[pl.ds(k * _CHUNK + p0, 16)] * _TSTRIDE
              for k in range(3)]
        sv = spos[pl.ds(p0, 16)]
        sp = jnp.where(sv == 0, 1, sv)
        sp = jnp.where(sp > 1, sp - 1, sp)
        sp = jnp.minimum(sp, MULTI_HOP_MAX_DIST)
        recip = (1.0 / 3.0) / sp.astype(jnp.float32)
        recip2 = plsc.pack(recip, recip, format=plsc.PackFormat.INTERLEAVED)
        svs = sv * _TSTRIDE
        for cp in range(NUM_HEADS // 2):
            cc = jnp.full((16,), cp, jnp.int32)
            s = ((_gather_bf(tbl_a, ia[0] + cc)
                  + _gather_bf(tbl_a, ia[1] + cc))
                 + _gather_bf(tbl_a, ia[2] + cc))
            spb = _gather_bf(tbl_b, svs + cc)
            slot = acc.at[cp, pl.ds(p0, 16)]
            prev = plsc.bitcast(slot[...], jnp.bfloat16)
            slot[...] = plsc.bitcast((prev + s) * recip2 + spb, jnp.int32)
        return carry

    lax.fori_loop(0, _GROUPS, sround, 0)

    pltpu.sync_copy(acc, out_hbm.at[:, pl.ds(base, _CHUNK)])


_sc_kernel = pl.kernel(
    _sc_body,
    out_type=jax.ShapeDtypeStruct((NUM_HEADS // 2, _NPAIR), jnp.int32),
    mesh=plsc.VectorSubcoreMesh(core_axis_name="c", subcore_axis_name="s"),
    compiler_params=pltpu.CompilerParams(needs_layout_passes=False),
    scratch_types=[
        pltpu.VMEM((_TSIZE,), jnp.int32),
        pltpu.VMEM((_TSIZE,), jnp.int32),
        pltpu.VMEM((NUM_HEADS // 2, _CHUNK), jnp.int32),
        pltpu.VMEM((6 * _CHUNK,), jnp.int32),
        pltpu.VMEM((_CHUNK,), jnp.int32),
        pltpu.SemaphoreType.DMA,
    ],
)


def _asm_body(ab_ref, rt_ref, t_ref, o_ref):
    ab2 = ab_ref[0] * 2.0
    w = rt_ref[...]
    lo = lax.bitcast_convert_type(w << 16, jnp.float32)
    hi = lax.bitcast_convert_type(w & jnp.int32(-65536), jnp.float32)
    inner = jnp.concatenate([lo, hi], axis=0).reshape(
        NUM_HEADS, N_NODE, N_NODE)
    t = t_ref[0]
    n1 = N_NODE + 1
    ii = lax.broadcasted_iota(jnp.int32, (NUM_HEADS, n1, n1), 1)
    jj = lax.broadcasted_iota(jnp.int32, (NUM_HEADS, n1, n1), 2)
    border = (ii == 0) | (jj == 0)
    tb = jnp.where(border,
                   jnp.broadcast_to(t[:, None, None], (NUM_HEADS, n1, n1)),
                   jnp.zeros((NUM_HEADS, n1, n1), jnp.float32))
    z_col = jnp.zeros((NUM_HEADS, N_NODE, 1), jnp.float32)
    z_row = jnp.zeros((NUM_HEADS, 1, n1), jnp.float32)
    padded = jnp.concatenate(
        [z_row, jnp.concatenate([z_col, inner], axis=2)], axis=1)
    o_ref[0] = ab2[None] + tb + padded


def _tc_asm(attn_bias, rt, gtvd_w):
    n1 = N_NODE + 1
    return pl.pallas_call(
        _asm_body,
        grid=(N_GRAPH,),
        in_specs=[
            pl.BlockSpec((1, n1, n1), lambda g: (g, 0, 0)),
            pl.BlockSpec((NUM_HEADS // 2, N_NODE * N_NODE), lambda g: (0, g)),
            pl.BlockSpec((1, NUM_HEADS), lambda g: (0, 0)),
        ],
        out_specs=pl.BlockSpec((1, NUM_HEADS, n1, n1), lambda g: (g, 0, 0, 0)),
        out_shape=jax.ShapeDtypeStruct(
            (N_GRAPH, NUM_HEADS, n1, n1), jnp.float32),
    )(attn_bias, rt, gtvd_w)


def kernel(attn_bias, node_attr, is_molecule, spatial_pos, edge_input,
           spatial_pos_w, gtvd_w, edge_enc_w, edge_dis_w):
    w = edge_dis_w.reshape(-1, EDGE_HIDDEN, NUM_HEADS)[:MULTI_HOP_MAX_DIST]
    eew_pad = jnp.zeros((_TROWS_P, EDGE_HIDDEN), jnp.float32)
    eew_pad = eew_pad.at[:_TBL_ROWS].set(edge_enc_w)
    t5 = _tc_pre(eew_pad, w)

    spw_pad = jnp.zeros((_TROWS_P, NUM_HEADS), jnp.float32)
    spw_pad = spw_pad.at[:spatial_pos_w.shape[0]].set(spatial_pos_w)
    all6 = jnp.concatenate([t5, spw_pad[None]], axis=0)
    lo = lax.bitcast_convert_type(
        all6[..., :16].astype(jnp.bfloat16), jnp.uint16).astype(jnp.uint32)
    hi = lax.bitcast_convert_type(
        all6[..., 16:].astype(jnp.bfloat16), jnp.uint16).astype(jnp.uint32)
    packed = lax.bitcast_convert_type(lo | (hi << 16), jnp.int32)
    packed = jnp.pad(packed, ((0, 0), (0, 0), (0, 1)))
    packed = packed.reshape(6 * _TSIZE)

    idx_t = jnp.swapaxes(edge_input.reshape(_NPAIR, _NSLOT), 0, 1)
    idx_t = idx_t.reshape(_NSLOT * _NPAIR)
    spos_flat = spatial_pos.reshape(_NPAIR)

    rt = _sc_kernel(packed, idx_t, spos_flat)
    return _tc_asm(attn_bias, rt, gtvd_w)


# speedup vs baseline: 1.2356x; 1.0253x over previous
import jax
import jax.numpy as jnp
from jax import lax
from jax.experimental import pallas as pl
from jax.experimental.pallas import tpu as pltpu
from jax.experimental.pallas import tpu_sc as plsc

NUM_HEADS = 32
EDGE_HIDDEN = 32
MULTI_HOP_MAX_DIST = 5
NUM_EDGES = 1536
N_GRAPH = 16
N_NODE = 64

_NPAIR = N_GRAPH * N_NODE * N_NODE
_TBL_ROWS = NUM_EDGES + 1
_TROWS_P = 1544
_TSTRIDE = 17
_TSIZE = _TROWS_P * _TSTRIDE
_SP_ROWS = 528
_SP_SIZE = _SP_ROWS * _TSTRIDE

_info = plsc.get_sparse_core_info()
_NC, _NS = _info.num_cores, _info.num_subcores
_NW = _NC * _NS
_CHUNK = _NPAIR // _NW
_GROUPS = _CHUNK // 16


def _tpre_body(e_ref, w_ref, o_ref):
    o_ref[0] = jnp.dot(e_ref[...], w_ref[0], preferred_element_type=jnp.float32)


def _tc_pre(edge_enc_w_pad, w):
    return pl.pallas_call(
        _tpre_body,
        grid=(MULTI_HOP_MAX_DIST,),
        in_specs=[
            pl.BlockSpec((_TROWS_P, EDGE_HIDDEN), lambda d: (0, 0)),
            pl.BlockSpec((1, EDGE_HIDDEN, NUM_HEADS), lambda d: (d, 0, 0)),
        ],
        out_specs=pl.BlockSpec((1, _TROWS_P, NUM_HEADS), lambda d: (d, 0, 0)),
        out_shape=jax.ShapeDtypeStruct(
            (MULTI_HOP_MAX_DIST, _TROWS_P, NUM_HEADS), jnp.float32),
    )(edge_enc_w_pad, w)


_NSLOT = MULTI_HOP_MAX_DIST * 3


def _tr_body(e_ref, o_ref):
    o_ref[...] = e_ref[...].T


def _tc_tr(edge_idx_2d):
    return pl.pallas_call(
        _tr_body,
        grid=(_NW,),
        in_specs=[pl.BlockSpec((_CHUNK, _NSLOT), lambda g: (g, 0))],
        out_specs=pl.BlockSpec((_NSLOT, _CHUNK), lambda g: (0, g)),
        out_shape=jax.ShapeDtypeStruct((_NSLOT, _NPAIR), jnp.int32),
    )(edge_idx_2d)


def _gather_bf(tbl, addr):
    return plsc.bitcast(plsc.load_gather(tbl, [addr]), jnp.bfloat16)


def _sc_body(tp_hbm, spp_hbm, idx_hbm, spos_hbm, out_hbm,
             tbl_a, tbl_b, spt, acc, idx15, spos, sem):
    wid = lax.axis_index("s") * _NC + lax.axis_index("c")
    base = wid * _CHUNK

    def tbl_dmas(da, db):
        cps = [pltpu.async_copy(
            tp_hbm.at[pl.ds(da * _TSIZE, _TSIZE)], tbl_a, sem)]
        if db is not None:
            cps.append(pltpu.async_copy(
                tp_hbm.at[pl.ds(db * _TSIZE, _TSIZE)], tbl_b, sem))
        return cps

    start = tbl_dmas(0, 1)
    start.append(pltpu.async_copy(spp_hbm, spt, sem))
    start.append(pltpu.async_copy(spos_hbm.at[pl.ds(base, _CHUNK)],
                                  spos, sem))
    for s in range(_NSLOT):
        start.append(pltpu.async_copy(
            idx_hbm.at[pl.ds(s * _NPAIR + base, _CHUNK)],
            idx15.at[pl.ds(s * _CHUNK, _CHUNK)], sem))
    for c in start:
        c.wait()

    for rnd, (da, db) in enumerate(((0, 1), (2, 3))):
        if rnd > 0:
            for c in tbl_dmas(da, db):
                c.wait()

        def eround(g, carry, first=(rnd == 0), da=da, db=db):
            p0 = pl.multiple_of(g * 16, 16)
            ia = [idx15[pl.ds((da * 3 + k) * _CHUNK + p0, 16)] * _TSTRIDE
                  for k in range(3)]
            ib = [idx15[pl.ds((db * 3 + k) * _CHUNK + p0, 16)] * _TSTRIDE
                  for k in range(3)]
            for cp in range(NUM_HEADS // 2):
                cc = jnp.full((16,), cp, jnp.int32)
                sa = (_gather_bf(tbl_a, ia[0] + cc)
                      + _gather_bf(tbl_a, ia[1] + cc))
                sb = (_gather_bf(tbl_a, ia[2] + cc)
                      + _gather_bf(tbl_b, ib[0] + cc))
                sc = (_gather_bf(tbl_b, ib[1] + cc)
                      + _gather_bf(tbl_b, ib[2] + cc))
                s = sa + sb + sc
                slot = acc.at[cp, pl.ds(p0, 16)]
                if first:
                    slot[...] = plsc.bitcast(s, jnp.int32)
                else:
                    prev = plsc.bitcast(slot[...], jnp.bfloat16)
                    slot[...] = plsc.bitcast(prev + s, jnp.int32)
            return carry

        lax.fori_loop(0, _GROUPS, eround, 0)

    for c in tbl_dmas(4, None):
        c.wait()

    def sround(g, carry):
        p0 = pl.multiple_of(g * 16, 16)
        ia = [idx15[pl.ds((12 + k) * _CHUNK + p0, 16)] * _TSTRIDE
              for k in range(3)]
        sv = spos[pl.ds(p0, 16)]
        sp = jnp.where(sv == 0, 1, sv)
        sp = jnp.where(sp > 1, sp - 1, sp)
        sp = jnp.minimum(sp, MULTI_HOP_MAX_DIST)
        recip = (1.0 / 3.0) / sp.astype(jnp.float32)
        recip2 = plsc.pack(recip, recip, format=plsc.PackFormat.INTERLEAVED)
        svs = sv * _TSTRIDE
        for cp in range(NUM_HEADS // 2):
            cc = jnp.full((16,), cp, jnp.int32)
            s = ((_gather_bf(tbl_a, ia[0] + cc)
                  + _gather_bf(tbl_a, ia[1] + cc))
                 + _gather_bf(tbl_a, ia[2] + cc))
            spb = _gather_bf(spt, svs + cc)
            slot = acc.at[cp, pl.ds(p0, 16)]
            prev = plsc.bitcast(slot[...], jnp.bfloat16)
            slot[...] = plsc.bitcast((prev + s) * recip2 + spb, jnp.int32)
        return carry

    lax.fori_loop(0, _GROUPS, sround, 0)

    pltpu.sync_copy(acc, out_hbm.at[:, pl.ds(base, _CHUNK)])


_sc_kernel = pl.kernel(
    _sc_body,
    out_type=jax.ShapeDtypeStruct((NUM_HEADS // 2, _NPAIR), jnp.int32),
    mesh=plsc.VectorSubcoreMesh(core_axis_name="c", subcore_axis_name="s"),
    compiler_params=pltpu.CompilerParams(needs_layout_passes=False),
    scratch_types=[
        pltpu.VMEM((_TSIZE,), jnp.int32),
        pltpu.VMEM((_TSIZE,), jnp.int32),
        pltpu.VMEM((_SP_SIZE,), jnp.int32),
        pltpu.VMEM((NUM_HEADS // 2, _CHUNK), jnp.int32),
        pltpu.VMEM((_NSLOT * _CHUNK,), jnp.int32),
        pltpu.VMEM((_CHUNK,), jnp.int32),
        pltpu.SemaphoreType.DMA,
    ],
)


def _asm_body(ab_ref, rt_ref, t_ref, o_ref):
    ab2 = ab_ref[0] * 2.0
    w = rt_ref[...]
    lo = lax.bitcast_convert_type(w << 16, jnp.float32)
    hi = lax.bitcast_convert_type(w & jnp.int32(-65536), jnp.float32)
    inner = jnp.concatenate([lo, hi], axis=0).reshape(
        NUM_HEADS, N_NODE, N_NODE)
    t = t_ref[0]
    n1 = N_NODE + 1
    ii = lax.broadcasted_iota(jnp.int32, (NUM_HEADS, n1, n1), 1)
    jj = lax.broadcasted_iota(jnp.int32, (NUM_HEADS, n1, n1), 2)
    border = (ii == 0) | (jj == 0)
    tb = jnp.where(border,
                   jnp.broadcast_to(t[:, None, None], (NUM_HEADS, n1, n1)),
                   jnp.zeros((NUM_HEADS, n1, n1), jnp.float32))
    z_col = jnp.zeros((NUM_HEADS, N_NODE, 1), jnp.float32)
    z_row = jnp.zeros((NUM_HEADS, 1, n1), jnp.float32)
    padded = jnp.concatenate(
        [z_row, jnp.concatenate([z_col, inner], axis=2)], axis=1)
    o_ref[0] = ab2[None] + tb + padded


def _tc_asm(attn_bias, rt, gtvd_w):
    n1 = N_NODE + 1
    return pl.pallas_call(
        _asm_body,
        grid=(N_GRAPH,),
        in_specs=[
            pl.BlockSpec((1, n1, n1), lambda g: (g, 0, 0)),
            pl.BlockSpec((NUM_HEADS // 2, N_NODE * N_NODE), lambda g: (0, g)),
            pl.BlockSpec((1, NUM_HEADS), lambda g: (0, 0)),
        ],
        out_specs=pl.BlockSpec((1, NUM_HEADS, n1, n1), lambda g: (g, 0, 0, 0)),
        out_shape=jax.ShapeDtypeStruct(
            (N_GRAPH, NUM_HEADS, n1, n1), jnp.float32),
    )(attn_bias, rt, gtvd_w)


def kernel(attn_bias, node_attr, is_molecule, spatial_pos, edge_input,
           spatial_pos_w, gtvd_w, edge_enc_w, edge_dis_w):
    w = edge_dis_w.reshape(-1, EDGE_HIDDEN, NUM_HEADS)[:MULTI_HOP_MAX_DIST]
    eew_pad = jnp.zeros((_TROWS_P, EDGE_HIDDEN), jnp.float32)
    eew_pad = eew_pad.at[:_TBL_ROWS].set(edge_enc_w)
    t5 = _tc_pre(eew_pad, w)

    def _pack_tbl(t):
        lo = lax.bitcast_convert_type(
            t[..., :16].astype(jnp.bfloat16), jnp.uint16).astype(jnp.uint32)
        hi = lax.bitcast_convert_type(
            t[..., 16:].astype(jnp.bfloat16), jnp.uint16).astype(jnp.uint32)
        p = lax.bitcast_convert_type(lo | (hi << 16), jnp.int32)
        widths = [(0, 0)] * (p.ndim - 1) + [(0, 1)]
        return jnp.pad(p, widths).reshape(-1)

    packed = _pack_tbl(t5)
    spw_pad = jnp.zeros((_SP_ROWS, NUM_HEADS), jnp.float32)
    spw_pad = spw_pad.at[:spatial_pos_w.shape[0]].set(spatial_pos_w)
    spp = _pack_tbl(spw_pad)

    idx_t = jnp.swapaxes(edge_input.reshape(_NPAIR, _NSLOT), 0, 1)
    idx_t = idx_t.reshape(_NSLOT * _NPAIR)
    spos_flat = spatial_pos.reshape(_NPAIR)

    rt = _sc_kernel(packed, spp, idx_t, spos_flat)
    return _tc_asm(attn_bias, rt, gtvd_w)
